# Initial kernel scaffold; baseline (speedup 1.0000x reference)
import jax
import jax.numpy as jnp
from jax.experimental import pallas as pl


def kernel(x, edge_index, W_in, b_in, W0, b0, W1, b1, W_out, b_out):
    raise NotImplementedError("write your pallas kernel here")


# speedup vs baseline: 6.7396x; 6.7396x over previous
import functools

import jax
import jax.numpy as jnp
from jax import lax
from jax.experimental import pallas as pl
from jax.experimental.pallas import tpu as pltpu
from jax.experimental.pallas import tpu_sc as plsc

N = 10000
E = 320000
D_IN = 128
S = 64
C = 40
BOUND = 10.0

NC = 2
NS = 16
NW = NC * NS
K = 128
EPW = -(-E // NW)
NCHUNK = -(-EPW // K)
EPW_PAD = NCHUNK * K
E_PAD = EPW_PAD * NW
NPAD = -(-N // (NS * 8)) * NS * 8
SLAB = NPAD // NS


def _tc_in_body(x_ref, w_ref, b_ref, o_ref):
    o_ref[...] = (
        jnp.dot(x_ref[...], w_ref[...], preferred_element_type=jnp.float32)
        + b_ref[...]
    )


def _tc_in(x, W_in, b_in):
    return pl.pallas_call(
        _tc_in_body,
        out_shape=jax.ShapeDtypeStruct((N, S), jnp.float32),
    )(x, W_in, b_in.reshape(1, S))


def _tc_update_body(h_ref, parts_ref, w_ref, b_ref, o_ref):
    agg = parts_ref[0:N, :] + parts_ref[NPAD:NPAD + N, :]
    clamped = jnp.clip(agg, 0.0, BOUND)
    o_ref[...] = (
        jnp.dot(h_ref[...], w_ref[0:S, :], preferred_element_type=jnp.float32)
        + jnp.dot(clamped, w_ref[S:2 * S, :], preferred_element_type=jnp.float32)
        + b_ref[...]
    )


def _tc_update(h, parts, W, b):
    return pl.pallas_call(
        _tc_update_body,
        out_shape=jax.ShapeDtypeStruct((N, S), jnp.float32),
    )(h, parts, W, b.reshape(1, S))


def _tc_out_body(h_ref, w_ref, b_ref, o_ref):
    logits = (
        jnp.dot(h_ref[...], w_ref[...], preferred_element_type=jnp.float32)
        + b_ref[...]
    )
    m = jnp.max(logits, axis=-1, keepdims=True)
    z = logits - m
    lse = jnp.log(jnp.sum(jnp.exp(z), axis=-1, keepdims=True))
    o_ref[...] = z - lse


def _tc_out(h, W_out, b_out):
    return pl.pallas_call(
        _tc_out_body,
        out_shape=jax.ShapeDtypeStruct((N, C), jnp.float32),
    )(h, W_out, b_out.reshape(1, C))


def _sc_body(h_hbm, src_hbm, dst_hbm, zeros_hbm, out_hbm,
             src_v, dst_v, rows_v, agg_sh, sem):
    c = lax.axis_index("c")
    s = lax.axis_index("s")
    wid = c * NS + s

    pltpu.sync_copy(zeros_hbm.at[pl.ds(s * SLAB, SLAB)],
                    agg_sh.at[pl.ds(s * SLAB, SLAB)])
    pltpu.sync_copy(src_hbm.at[wid], src_v)
    pltpu.sync_copy(dst_hbm.at[wid], dst_v)
    plsc.subcore_barrier()

    def chunk(j, carry):
        pltpu.async_copy(h_hbm.at[src_v.at[j]], rows_v, sem).wait()
        pltpu.sync_copy(rows_v, agg_sh.at[dst_v.at[j]], add=True)
        return carry

    lax.fori_loop(0, NCHUNK, chunk, 0, unroll=False)
    plsc.subcore_barrier()
    pltpu.sync_copy(agg_sh.at[pl.ds(s * SLAB, SLAB)],
                    out_hbm.at[pl.ds(c * NPAD + s * SLAB, SLAB)])


@functools.partial(
    pl.kernel,
    out_type=jax.ShapeDtypeStruct((NC * NPAD, S), jnp.float32),
    mesh=plsc.VectorSubcoreMesh(core_axis_name="c", subcore_axis_name="s",
                                num_cores=NC, num_subcores=NS),
    compiler_params=pltpu.CompilerParams(use_tc_tiling_on_sc=False),
    scratch_types=[
        pltpu.VMEM((NCHUNK, K), jnp.int32),
        pltpu.VMEM((NCHUNK, K), jnp.int32),
        pltpu.VMEM((K, S), jnp.float32),
        pltpu.VMEM_SHARED((NPAD, S), jnp.float32),
        pltpu.SemaphoreType.DMA,
    ],
)
def _sc_layer(*args):
    _sc_body(*args)


def kernel(x, edge_index, W_in, b_in, W0, b0, W1, b1, W_out, b_out):
    src = edge_index[0].astype(jnp.int32)
    dst = edge_index[1].astype(jnp.int32)
    pad = E_PAD - E
    src3 = jnp.concatenate([src, jnp.zeros((pad,), jnp.int32)])
    src3 = src3.reshape(NW, NCHUNK, K)
    dst3 = jnp.concatenate([dst, jnp.full((pad,), N, jnp.int32)])
    dst3 = dst3.reshape(NW, NCHUNK, K)
    zeros = jnp.zeros((NPAD, S), jnp.float32)

    h = _tc_in(x.astype(jnp.float32), W_in, b_in)
    parts = _sc_layer(h, src3, dst3, zeros)
    h = _tc_update(h, parts, W0, b0)
    parts = _sc_layer(h, src3, dst3, zeros)
    h = _tc_update(h, parts, W1, b1)
    return _tc_out(h, W_out, b_out)
